# Initial kernel scaffold; baseline (speedup 1.0000x reference)
import jax
import jax.numpy as jnp
from jax.experimental import pallas as pl


def kernel(X, Y, intercept):
    raise NotImplementedError("write your pallas kernel here")


# speedup vs baseline: 1.8388x; 1.8388x over previous
import jax
import jax.numpy as jnp
from jax.experimental import pallas as pl

_TEMP = 50.0
_BN = 256


def _fm_kernel(x_ref, y_ref, b_ref, choice_ref, fx_ref):
    x = x_ref[...]
    y = y_ref[...]
    b = b_ref[...]
    c = jnp.sum(y * y, axis=1) + b[0]
    xy = jax.lax.dot_general(x, y, (((1,), (1,)), ((), ())),
                             preferred_element_type=jnp.float32)
    s = 2.0 * xy - c[None, :]
    m = jnp.max(s, axis=1, keepdims=True)
    e = jnp.exp(_TEMP * (s - m))
    se = jnp.sum(e, axis=1, keepdims=True)
    x2 = jnp.sum(x * x, axis=1)
    fx_ref[...] = jnp.sum(e * s, axis=1) / se[:, 0] - x2
    choice_ref[...] = jax.lax.dot_general(e, y, (((1,), (0,)), ((), ())),
                                          preferred_element_type=jnp.float32) / se


def kernel(X, Y, intercept):
    N, D = X.shape
    K = Y.shape[1]
    choice, fx = pl.pallas_call(
        _fm_kernel,
        grid=(N // _BN,),
        in_specs=[
            pl.BlockSpec((_BN, D), lambda i: (i, 0)),
            pl.BlockSpec((K, D), lambda i: (0, 0)),
            pl.BlockSpec((1, K), lambda i: (0, 0)),
        ],
        out_specs=[
            pl.BlockSpec((_BN, D), lambda i: (i, 0)),
            pl.BlockSpec((_BN,), lambda i: (i,)),
        ],
        out_shape=[
            jax.ShapeDtypeStruct((N, D), jnp.float32),
            jax.ShapeDtypeStruct((N,), jnp.float32),
        ],
    )(X, Y[0], intercept)
    return choice, fx
